# Initial kernel scaffold; baseline (speedup 1.0000x reference)
import jax
import jax.numpy as jnp
from jax.experimental import pallas as pl


def kernel(predicate_ids, table):
    raise NotImplementedError("write your pallas kernel here")


# speedup vs baseline: 1.8446x; 1.8446x over previous
import functools

import jax
import jax.numpy as jnp
from jax import lax
from jax.experimental import pallas as pl
from jax.experimental.pallas import tpu as pltpu
from jax.experimental.pallas import tpu_sc as plsc

BATCH = 16384
HIST = 50
EMBED_DIM = 64

B = BATCH * HIST
NC = 2
NS = 16
NW = NC * NS
G = 128
NROWS = B // G
RPW = NROWS // NW
K = 8
NT = RPW // K

_mesh = plsc.VectorSubcoreMesh(core_axis_name="c", subcore_axis_name="s")


@functools.partial(
    pl.kernel,
    out_type=jax.ShapeDtypeStruct((NROWS, G, EMBED_DIM), jnp.float32),
    mesh=_mesh,
    scratch_types=[
        pltpu.VMEM((K, G), jnp.int32),
        pltpu.VMEM((K, G, EMBED_DIM), jnp.float32),
        pltpu.SemaphoreType.DMA,
    ],
    compiler_params=pltpu.CompilerParams(use_tc_tiling_on_sc=False),
)
def _sc_gather(table_hbm, idx_hbm, out_hbm, idx_v, rows_v, sem):
    wid = lax.axis_index("s") * NC + lax.axis_index("c")
    row0 = wid * RPW

    def step(t, carry):
        r0 = row0 + t * K
        pltpu.sync_copy(idx_hbm.at[pl.ds(r0, K)], idx_v)
        descs = [
            pltpu.async_copy(table_hbm.at[idx_v.at[j]], rows_v.at[j], sem)
            for j in range(K)
        ]
        for d in descs:
            d.wait()
        pltpu.sync_copy(rows_v, out_hbm.at[pl.ds(r0, K)])
        return carry

    lax.fori_loop(0, NT, step, 0)


def kernel(predicate_ids, table):
    idx = predicate_ids.astype(jnp.int32).reshape(NROWS, G)
    out = _sc_gather(table, idx)
    return out.reshape(BATCH, HIST, EMBED_DIM)


# speedup vs baseline: 1.8764x; 1.0172x over previous
import functools

import jax
import jax.numpy as jnp
from jax import lax
from jax.experimental import pallas as pl
from jax.experimental.pallas import tpu as pltpu
from jax.experimental.pallas import tpu_sc as plsc

BATCH = 16384
HIST = 50
EMBED_DIM = 64

B = BATCH * HIST
NC = 2
NS = 16
NW = NC * NS
G = 128
NROWS = B // G
RPW = NROWS // NW
NBUF = 10
GLEAD = 6
SLEAD = NBUF - GLEAD
NT = RPW // NBUF

_mesh = plsc.VectorSubcoreMesh(core_axis_name="c", subcore_axis_name="s")


@functools.partial(
    pl.kernel,
    out_type=jax.ShapeDtypeStruct((NROWS, G, EMBED_DIM), jnp.float32),
    mesh=_mesh,
    scratch_types=[
        pltpu.VMEM((RPW, G), jnp.int32),
        pltpu.VMEM((NBUF, G, EMBED_DIM), jnp.float32),
    ]
    + [pltpu.SemaphoreType.DMA] * (2 * NBUF),
    compiler_params=pltpu.CompilerParams(use_tc_tiling_on_sc=False),
)
def _sc_gather(table_hbm, idx_hbm, out_hbm, idx_v, rows_v, *sems):
    gsem = sems[:NBUF]
    ssem = sems[NBUF:]
    wid = lax.axis_index("s") * NC + lax.axis_index("c")
    row0 = wid * RPW

    pltpu.sync_copy(idx_hbm.at[pl.ds(row0, RPW)], idx_v)

    for b in range(GLEAD):
        pltpu.async_copy(table_hbm.at[idx_v.at[b]], rows_v.at[b], gsem[b])

    def step(t, carry):
        for b in range(NBUF):
            r = t * NBUF + b
            pltpu.make_async_copy(
                table_hbm.at[idx_v.at[0]], rows_v.at[b], gsem[b]
            ).wait()
            pltpu.async_copy(rows_v.at[b], out_hbm.at[row0 + r], ssem[b])
            bn = (b + GLEAD) % NBUF

            @pl.when(r >= SLEAD)
            def _():
                pltpu.make_async_copy(
                    rows_v.at[bn], out_hbm.at[0], ssem[bn]
                ).wait()

            @pl.when(r + GLEAD < RPW)
            def _():
                pltpu.async_copy(
                    table_hbm.at[idx_v.at[r + GLEAD]], rows_v.at[bn], gsem[bn]
                )

        return carry

    lax.fori_loop(0, NT, step, 0)

    for b in range(GLEAD, NBUF):
        pltpu.make_async_copy(rows_v.at[b], out_hbm.at[0], ssem[b]).wait()


def kernel(predicate_ids, table):
    idx = predicate_ids.astype(jnp.int32).reshape(NROWS, G)
    out = _sc_gather(table, idx)
    return out.reshape(BATCH, HIST, EMBED_DIM)
